# Initial kernel scaffold; baseline (speedup 1.0000x reference)
import jax
import jax.numpy as jnp
from jax.experimental import pallas as pl


def kernel(x, edge_index, batch, climber, W1, a_src1, a_dst1, b1, W2, a_src2, a_dst2, b2, Wc, bc, Wf1, bf1, Wf2, bf2):
    raise NotImplementedError("write your pallas kernel here")


# speedup vs baseline: 84.4469x; 84.4469x over previous
import functools

import jax
import jax.numpy as jnp
from jax import lax
from jax.experimental import pallas as pl
from jax.experimental.pallas import tpu as pltpu
from jax.experimental.pallas import tpu_sc as plsc

N = 100000
E = 1600000
G = 1024
HID = 64
F = 2 * HID
B = 20
RPB = N // B
TRASH = 16
SROWS = RPB + TRASH
NC, NS = 2, 16
NW = NC * NS
EPW = E // NW
CH = 2000
NSTEP = EPW // CH
CAPW = 2944
CAPA = CAPW + 16
K = 128

_f32 = jnp.float32
_i32 = jnp.int32


def _lrelu(v):
    return jnp.where(v >= 0, v, 0.2 * v)


def _prep_body(hin_ref, w_ref, as_ref, ad_ref, feat_ref, as0_ref, as1_ref,
               ad0_ref, ad1_ref, selfw_ref, self_ref):
    h = jnp.dot(hin_ref[...], w_ref[...], preferred_element_type=_f32)
    a_s = as_ref[...]
    a_d = ad_ref[...]
    as0 = jnp.sum(h[:, :HID] * a_s[0:1, :], axis=1, keepdims=True)
    as1 = jnp.sum(h[:, HID:] * a_s[1:2, :], axis=1, keepdims=True)
    ad0 = jnp.sum(h[:, :HID] * a_d[0:1, :], axis=1, keepdims=True)
    ad1 = jnp.sum(h[:, HID:] * a_d[1:2, :], axis=1, keepdims=True)
    feat_ref[...] = h
    as0_ref[...] = as0
    as1_ref[...] = as1
    ad0_ref[...] = ad0
    ad1_ref[...] = ad1
    w0 = jnp.exp(_lrelu(as0 + ad0))
    w1 = jnp.exp(_lrelu(as1 + ad1))
    selfw_ref[...] = jnp.concatenate([w0, w1], axis=1)
    self_ref[...] = jnp.concatenate(
        [h[:, :HID] * w0, h[:, HID:] * w1], axis=1)


def _prep(hin, w, a_s, a_d):
    blk = 2000
    grid = N // blk
    kin = hin.shape[1]
    return pl.pallas_call(
        _prep_body,
        grid=(grid,),
        in_specs=[
            pl.BlockSpec((blk, kin), lambda i: (i, 0)),
            pl.BlockSpec((kin, F), lambda i: (0, 0)),
            pl.BlockSpec((2, HID), lambda i: (0, 0)),
            pl.BlockSpec((2, HID), lambda i: (0, 0)),
        ],
        out_specs=[
            pl.BlockSpec((blk, F), lambda i: (i, 0)),
            pl.BlockSpec((blk, 1), lambda i: (i, 0)),
            pl.BlockSpec((blk, 1), lambda i: (i, 0)),
            pl.BlockSpec((blk, 1), lambda i: (i, 0)),
            pl.BlockSpec((blk, 1), lambda i: (i, 0)),
            pl.BlockSpec((blk, 2), lambda i: (i, 0)),
            pl.BlockSpec((blk, F), lambda i: (i, 0)),
        ],
        out_shape=[
            jax.ShapeDtypeStruct((N, F), _f32),
            jax.ShapeDtypeStruct((N, 1), _f32),
            jax.ShapeDtypeStruct((N, 1), _f32),
            jax.ShapeDtypeStruct((N, 1), _f32),
            jax.ShapeDtypeStruct((N, 1), _f32),
            jax.ShapeDtypeStruct((N, 2), _f32),
            jax.ShapeDtypeStruct((N, F), _f32),
        ],
    )(hin, w, a_s, a_d)


def _mean_body(acc_ref, den0_ref, den1_ref, selfw_ref, b_ref, out_ref):
    acc = acc_ref[...]
    selfw = selfw_ref[...]
    den0 = den0_ref[...] + selfw[:, 0:1] + 1e-16
    den1 = den1_ref[...] + selfw[:, 1:2] + 1e-16
    m = 0.5 * (acc[:, :HID] / den0 + acc[:, HID:F] / den1) + b_ref[...]
    out_ref[...] = jnp.maximum(m, 0.0)


def _head_mean(acc, den0, den1, selfw, bvec):
    blk = 2000
    return pl.pallas_call(
        _mean_body,
        grid=(N // blk,),
        in_specs=[
            pl.BlockSpec((blk, F), lambda i: (i, 0)),
            pl.BlockSpec((blk, 1), lambda i: (i, 0)),
            pl.BlockSpec((blk, 1), lambda i: (i, 0)),
            pl.BlockSpec((blk, 2), lambda i: (i, 0)),
            pl.BlockSpec((1, HID), lambda i: (0, 0)),
        ],
        out_specs=pl.BlockSpec((blk, HID), lambda i: (i, 0)),
        out_shape=jax.ShapeDtypeStruct((N, HID), _f32),
    )(acc, den0.reshape(N, 1), den1.reshape(N, 1), selfw, bvec)


def _final_body(h_ref, batch_ref, t_ref, wf1_ref, bf1_ref, wf2_ref, bf2_ref,
                out_ref):
    h = h_ref[...]
    z1 = jnp.dot(h, wf1_ref[...][:HID, :], preferred_element_type=_f32)
    bb = batch_ref[...]
    gids = lax.broadcasted_iota(_i32, (bb.shape[0], G), 1)
    oh = (bb == gids).astype(_f32)
    z2 = jnp.dot(oh, t_ref[...], preferred_element_type=_f32)
    z = jnp.maximum(z1 + z2 + bf1_ref[...], 0.0)
    out_ref[...] = jnp.dot(z, wf2_ref[...], preferred_element_type=_f32) \
        + bf2_ref[...]


def _final(hmean, batch2d, t, wf1, bf1v, wf2, bf2v):
    blk = 2000
    return pl.pallas_call(
        _final_body,
        grid=(N // blk,),
        in_specs=[
            pl.BlockSpec((blk, HID), lambda i: (i, 0)),
            pl.BlockSpec((blk, 1), lambda i: (i, 0)),
            pl.BlockSpec((G, HID), lambda i: (0, 0)),
            pl.BlockSpec((F, HID), lambda i: (0, 0)),
            pl.BlockSpec((1, HID), lambda i: (0, 0)),
            pl.BlockSpec((HID, 4), lambda i: (0, 0)),
            pl.BlockSpec((1, 4), lambda i: (0, 0)),
        ],
        out_specs=pl.BlockSpec((blk, 4), lambda i: (i, 0)),
        out_shape=jax.ShapeDtypeStruct((N, 4), _f32),
    )(hmean, batch2d, t, wf1, bf1v, wf2, bf2v)


def _t_body(c_ref, wc_ref, bc_ref, wf1_ref, t_ref):
    c = jnp.maximum(
        jnp.dot(c_ref[...], wc_ref[...], preferred_element_type=_f32)
        + bc_ref[...], 0.0)
    t_ref[...] = jnp.dot(c, wf1_ref[...][HID:, :], preferred_element_type=_f32)


def _climber_t(climber, wc, bcv, wf1):
    return pl.pallas_call(
        _t_body,
        out_shape=jax.ShapeDtypeStruct((G, HID), _f32),
    )(climber, wc, bcv, wf1)


def _bucketize_body(src_hbm, dst_hbm, bsrc_hbm, bdst_hbm, cnt_hbm,
                    svm, dvm, cvm, *bufs):
    sbufs = bufs[:B]
    dbufs = bufs[B:]
    wid = lax.axis_index("s") * NC + lax.axis_index("c")
    lane = lax.iota(_i32, 16)

    for b in range(B):
        psrc0 = (wid * 16 + lane) * 97 + b * 131
        pdst = jnp.broadcast_to(b * RPB + RPB + (wid % 16), (16,))

        def _fill(j, _, b=b, psrc0=psrc0, pdst=pdst):
            sbufs[b][pl.ds(j * 16, 16)] = psrc0 + j * 7
            dbufs[b][pl.ds(j * 16, 16)] = pdst
            return 0
        lax.fori_loop(0, CAPA // 16, _fill, 0)

    def step_body(step, pos):
        base = wid * EPW + step * CH
        pltpu.sync_copy(src_hbm.at[pl.ds(base, CH)], svm)
        pltpu.sync_copy(dst_hbm.at[pl.ds(base, CH)], dvm)

        def it_body(i, pos):
            sv = svm[pl.ds(i * 16, 16)]
            dv = dvm[pl.ds(i * 16, 16)]
            newpos = []
            for b in range(B):
                m = (dv >= b * RPB) & (dv < (b + 1) * RPB)
                cs = jnp.cumsum(m.astype(_i32))
                idx = pos[b] + cs - 1
                plsc.store_scatter(sbufs[b], [idx], sv, mask=m)
                plsc.store_scatter(dbufs[b], [idx], dv, mask=m)
                newpos.append(pos[b] + plsc.all_reduce_population_count(m))
            return tuple(newpos)
        return lax.fori_loop(0, CH // 16, it_body, pos)

    pos = lax.fori_loop(0, NSTEP, step_body,
                        tuple(jnp.zeros((16,), _i32) for _ in range(B)))

    cvecA = jnp.zeros((16,), _i32)
    cvecB = jnp.zeros((16,), _i32)
    for b in range(B):
        cpad = (pos[b] + (K - 1)) & ~(K - 1)
        if b < 16:
            cvecA = jnp.where(lane == b, cpad, cvecA)
        else:
            cvecB = jnp.where(lane == b - 16, cpad, cvecB)
    cvm[pl.ds(0, 16)] = cvecA
    cvm[pl.ds(16, 16)] = cvecB
    pltpu.sync_copy(cvm, cnt_hbm.at[wid])
    for b in range(B):
        cell = (b * NW + wid) * CAPW
        pltpu.sync_copy(sbufs[b].at[pl.ds(0, CAPW)],
                        bsrc_hbm.at[pl.ds(cell, CAPW)])
        pltpu.sync_copy(dbufs[b].at[pl.ds(0, CAPW)],
                        bdst_hbm.at[pl.ds(cell, CAPW)])


def _bucketize(src, dst):
    mesh = plsc.VectorSubcoreMesh(core_axis_name="c", subcore_axis_name="s")
    scratch = [
        pltpu.VMEM((CH,), _i32),
        pltpu.VMEM((CH,), _i32),
        pltpu.VMEM((32,), _i32),
    ] + [pltpu.VMEM((CAPA,), _i32) for _ in range(2 * B)]
    return pl.kernel(
        _bucketize_body,
        out_type=[
            jax.ShapeDtypeStruct((B * NW * CAPW,), _i32),
            jax.ShapeDtypeStruct((B * NW * CAPW,), _i32),
            jax.ShapeDtypeStruct((NW, 32), _i32),
        ],
        mesh=mesh,
        scratch_types=scratch,
        compiler_params=pltpu.CompilerParams(needs_layout_passes=False),
    )(src, dst)


RPT = 312
TAIL = RPB - NS * RPT


def _agg_body(feat_hbm, as0_hbm, as1_hbm, ad0_hbm, ad1_hbm, self_hbm,
              bsrc_hbm, bdst_hbm, cnt_hbm,
              acc_hbm, den0_hbm, den1_hbm, dscr_hbm,
              acc_sh, srcv, dstv, dlv, fbuf, mbuf, s0b, s1b,
              ad0v, ad1v, den0v, den1v, dtmp, dacc, dtmp2, dacc2, cvm,
              sem0, sem1, sem2):
    c = lax.axis_index("c")
    s = lax.axis_index("s")
    lane = lax.iota(_i32, 16)
    z16 = jnp.zeros((16,), _f32)

    pltpu.sync_copy(cnt_hbm, cvm)

    def bucket_body(k, _):
        b = NC * k + c
        lo = b * RPB
        pltpu.sync_copy(ad0_hbm.at[pl.ds(lo, SROWS)],
                        ad0v.at[pl.ds(0, SROWS)])
        pltpu.sync_copy(ad1_hbm.at[pl.ds(lo, SROWS)],
                        ad1v.at[pl.ds(0, SROWS)])

        def _zero(i, _):
            den0v[pl.ds(i * 16, 16)] = z16
            den1v[pl.ds(i * 16, 16)] = z16
            return 0
        lax.fori_loop(0, 5024 // 16, _zero, 0)

        pltpu.sync_copy(
            self_hbm.at[pl.ds(lo + s * RPT, RPT)],
            acc_sh.at[pl.ds(s * RPT, RPT)])

        @pl.when(s == NS - 1)
        def _():
            pltpu.sync_copy(
                self_hbm.at[pl.ds(lo + NS * RPT, TAIL)],
                acc_sh.at[pl.ds(NS * RPT, TAIL)])
        plsc.subcore_barrier()

        for j in range(2):
            w = 2 * s + j
            crowA = cvm[w, pl.ds(0, 16)]
            crowB = cvm[w, pl.ds(16, 16)]
            cnt = jnp.sum(jnp.where(lane == b, crowA, 0)
                          + jnp.where(lane == b - 16, crowB, 0))
            cell = (b * NW + w) * CAPW

            def chunk_body(ci, _, cell=cell, lo=lo):
                off = cell + ci * K
                pltpu.sync_copy(bsrc_hbm.at[pl.ds(off, K)], srcv)
                pltpu.sync_copy(bdst_hbm.at[pl.ds(off, K)], dstv)
                d0 = pltpu.async_copy(feat_hbm.at[srcv], fbuf, sem0)
                d1 = pltpu.async_copy(as0_hbm.at[srcv], s0b, sem1)
                d2 = pltpu.async_copy(as1_hbm.at[srcv], s1b, sem2)
                for q in range(K // 16):
                    dlv[pl.ds(q * 16, 16)] = dstv[pl.ds(q * 16, 16)] - lo
                d1.wait()
                d2.wait()
                d0.wait()

                def grp_body(g, _):
                    dlg = dlv[pl.ds(g * 16, 16)]
                    a0 = plsc.load_gather(ad0v, [dlg])
                    a1 = plsc.load_gather(ad1v, [dlg])
                    w0v = jnp.exp(_lrelu(s0b[pl.ds(g * 16, 16)] + a0))
                    w1v = jnp.exp(_lrelu(s1b[pl.ds(g * 16, 16)] + a1))
                    plsc.addupdate_scatter(den0v, [dlg], w0v)
                    plsc.addupdate_scatter(den1v, [dlg], w1v)
                    for j2 in range(16):
                        w0 = w0v[j2]
                        w1 = w1v[j2]
                        i = g * 16 + j2
                        for q in range(4):
                            mbuf[i, pl.ds(q * 16, 16)] = \
                                w0 * fbuf[i, pl.ds(q * 16, 16)]
                        for q in range(4, 8):
                            mbuf[i, pl.ds(q * 16, 16)] = \
                                w1 * fbuf[i, pl.ds(q * 16, 16)]
                    return 0
                lax.fori_loop(0, K // 16, grp_body, 0)
                pltpu.sync_copy(mbuf, acc_sh.at[dlv], add=True)
                return 0
            lax.fori_loop(0, cnt // K, chunk_body, 0)

        dbase = c * 2 * NS * SROWS
        for h in range(2):
            dv = den0v if h == 0 else den1v
            dhbm = den0_hbm if h == 0 else den1_hbm
            hbase = dbase + h * NS * SROWS
            pltpu.sync_copy(dv.at[pl.ds(0, SROWS)],
                            dscr_hbm.at[pl.ds(hbase + s * SROWS, SROWS)])
        plsc.subcore_barrier()
        for h in range(2):
            dhbm = den0_hbm if h == 0 else den1_hbm
            hbase = dbase + h * NS * SROWS

            def _z2(i, _):
                dacc[pl.ds(i * 16, 16)] = z16
                return 0
            lax.fori_loop(0, 320 // 16, _z2, 0)
            for t in range(NS):
                pltpu.sync_copy(
                    dscr_hbm.at[pl.ds(hbase + t * SROWS + s * RPT, RPT)],
                    dtmp.at[pl.ds(0, RPT)])

                def _acc(i, _):
                    dacc[pl.ds(i * 16, 16)] = (dacc[pl.ds(i * 16, 16)]
                                               + dtmp[pl.ds(i * 16, 16)])
                    return 0
                lax.fori_loop(0, 320 // 16, _acc, 0)
            pltpu.sync_copy(dacc.at[pl.ds(0, RPT)],
                            dhbm.at[pl.ds(lo + s * RPT, RPT)])

            @pl.when(s == NS - 1)
            def _():
                dacc2[...] = z16
                for t in range(NS):
                    pltpu.sync_copy(
                        dscr_hbm.at[pl.ds(hbase + t * SROWS + NS * RPT,
                                          TAIL)],
                        dtmp2.at[pl.ds(0, TAIL)])
                    dacc2[...] = dacc2[...] + dtmp2[...]
                pltpu.sync_copy(dacc2.at[pl.ds(0, TAIL)],
                                dhbm.at[pl.ds(lo + NS * RPT, TAIL)])

        pltpu.sync_copy(
            acc_sh.at[pl.ds(s * RPT, RPT)],
            acc_hbm.at[pl.ds(lo + s * RPT, RPT)])

        @pl.when(s == NS - 1)
        def _():
            pltpu.sync_copy(
                acc_sh.at[pl.ds(NS * RPT, TAIL)],
                acc_hbm.at[pl.ds(lo + NS * RPT, TAIL)])
        plsc.subcore_barrier()
        return 0

    lax.fori_loop(0, B // NC, bucket_body, 0)


def _aggregate(feat, as0, as1, ad0, ad1, selfinit, bsrc, bdst, cnt):
    mesh = plsc.VectorSubcoreMesh(core_axis_name="c", subcore_axis_name="s")
    scratch = [
        pltpu.VMEM_SHARED((SROWS, F), _f32),
        pltpu.VMEM((K,), _i32),
        pltpu.VMEM((K,), _i32),
        pltpu.VMEM((K,), _i32),
        pltpu.VMEM((K, F), _f32),
        pltpu.VMEM((K, F), _f32),
        pltpu.VMEM((K,), _f32),
        pltpu.VMEM((K,), _f32),
        pltpu.VMEM((5024,), _f32),
        pltpu.VMEM((5024,), _f32),
        pltpu.VMEM((5024,), _f32),
        pltpu.VMEM((5024,), _f32),
        pltpu.VMEM((320,), _f32),
        pltpu.VMEM((320,), _f32),
        pltpu.VMEM((16,), _f32),
        pltpu.VMEM((16,), _f32),
        pltpu.VMEM((NW, 32), _i32),
        pltpu.SemaphoreType.DMA,
        pltpu.SemaphoreType.DMA,
        pltpu.SemaphoreType.DMA,
    ]
    return pl.kernel(
        _agg_body,
        out_type=[
            jax.ShapeDtypeStruct((N, F), _f32),
            jax.ShapeDtypeStruct((N,), _f32),
            jax.ShapeDtypeStruct((N,), _f32),
            jax.ShapeDtypeStruct((NC * 2 * NS * SROWS,), _f32),
        ],
        mesh=mesh,
        scratch_types=scratch,
        compiler_params=pltpu.CompilerParams(needs_layout_passes=False),
    )(feat, as0, as1, ad0, ad1, selfinit, bsrc, bdst, cnt)


def kernel(x, edge_index, batch, climber, W1, a_src1, a_dst1, b1,
           W2, a_src2, a_dst2, b2, Wc, bc, Wf1, bf1, Wf2, bf2):
    src = edge_index[0]
    dst = edge_index[1]
    bsrc, bdst, cnt = _bucketize(src, dst)

    def _layer(hin, w, a_s, a_d, bvec):
        feat, as0, as1, ad0, ad1, selfw, selfi = _prep(
            hin, w, a_s.reshape(2, HID), a_d.reshape(2, HID))
        as0 = as0.reshape(N)
        as1 = as1.reshape(N)
        ad0 = jnp.pad(ad0.reshape(N), (0, TRASH))
        ad1 = jnp.pad(ad1.reshape(N), (0, TRASH))
        acc, den0, den1, _ = _aggregate(feat, as0, as1, ad0, ad1, selfi,
                                        bsrc, bdst, cnt)
        return _head_mean(acc, den0, den1, selfw, bvec.reshape(1, HID))

    h1 = _layer(x, W1, a_src1, a_dst1, b1)
    h2 = _layer(h1, W2, a_src2, a_dst2, b2)

    t = _climber_t(climber, Wc, bc.reshape(1, HID), Wf1)
    return _final(h2, batch.reshape(N, 1), t, Wf1, bf1.reshape(1, HID),
                  Wf2, bf2.reshape(1, 4))


# speedup vs baseline: 105.5818x; 1.2503x over previous
import jax
import jax.numpy as jnp
from jax import lax
from jax.experimental import pallas as pl
from jax.experimental.pallas import tpu as pltpu
from jax.experimental.pallas import tpu_sc as plsc

N = 100000
E = 1600000
G = 1024
HID = 64
F = 2 * HID
B = 20
RPB = N // B
TRASH = 16
SROWS = RPB + TRASH
NC, NS = 2, 16
NW = NC * NS
EPW = E // NW
CH = 1000
NSTEP = EPW // CH
CAPW = 3072
CAPA = CAPW + 16
K = 128

_f32 = jnp.float32
_i32 = jnp.int32


def _lrelu(v):
    return jnp.where(v >= 0, v, 0.2 * v)


def _prep_body(hin_ref, w_ref, as_ref, ad_ref, feat_ref, as0_ref, as1_ref,
               ad0_ref, ad1_ref, selfw_ref, self_ref):
    h = jnp.dot(hin_ref[...], w_ref[...], preferred_element_type=_f32)
    a_s = as_ref[...]
    a_d = ad_ref[...]
    as0 = jnp.sum(h[:, :HID] * a_s[0:1, :], axis=1, keepdims=True)
    as1 = jnp.sum(h[:, HID:] * a_s[1:2, :], axis=1, keepdims=True)
    ad0 = jnp.sum(h[:, :HID] * a_d[0:1, :], axis=1, keepdims=True)
    ad1 = jnp.sum(h[:, HID:] * a_d[1:2, :], axis=1, keepdims=True)
    feat_ref[...] = h
    as0_ref[...] = as0
    as1_ref[...] = as1
    ad0_ref[...] = ad0
    ad1_ref[...] = ad1
    w0 = jnp.exp(_lrelu(as0 + ad0))
    w1 = jnp.exp(_lrelu(as1 + ad1))
    selfw_ref[...] = jnp.concatenate([w0, w1], axis=1)
    self_ref[...] = jnp.concatenate(
        [h[:, :HID] * w0, h[:, HID:] * w1], axis=1)


def _prep(hin, w, a_s, a_d):
    blk = 2000
    grid = N // blk
    kin = hin.shape[1]
    return pl.pallas_call(
        _prep_body,
        grid=(grid,),
        in_specs=[
            pl.BlockSpec((blk, kin), lambda i: (i, 0)),
            pl.BlockSpec((kin, F), lambda i: (0, 0)),
            pl.BlockSpec((2, HID), lambda i: (0, 0)),
            pl.BlockSpec((2, HID), lambda i: (0, 0)),
        ],
        out_specs=[
            pl.BlockSpec((blk, F), lambda i: (i, 0)),
            pl.BlockSpec((blk, 1), lambda i: (i, 0)),
            pl.BlockSpec((blk, 1), lambda i: (i, 0)),
            pl.BlockSpec((blk, 1), lambda i: (i, 0)),
            pl.BlockSpec((blk, 1), lambda i: (i, 0)),
            pl.BlockSpec((blk, 2), lambda i: (i, 0)),
            pl.BlockSpec((blk, F), lambda i: (i, 0)),
        ],
        out_shape=[
            jax.ShapeDtypeStruct((N, F), _f32),
            jax.ShapeDtypeStruct((N, 1), _f32),
            jax.ShapeDtypeStruct((N, 1), _f32),
            jax.ShapeDtypeStruct((N, 1), _f32),
            jax.ShapeDtypeStruct((N, 1), _f32),
            jax.ShapeDtypeStruct((N, 2), _f32),
            jax.ShapeDtypeStruct((N, F), _f32),
        ],
    )(hin, w, a_s, a_d)


def _mean_body(acc_ref, den0_ref, den1_ref, selfw_ref, b_ref, out_ref):
    acc = acc_ref[...]
    selfw = selfw_ref[...]
    den0 = den0_ref[...] + selfw[:, 0:1] + 1e-16
    den1 = den1_ref[...] + selfw[:, 1:2] + 1e-16
    m = 0.5 * (acc[:, :HID] / den0 + acc[:, HID:F] / den1) + b_ref[...]
    out_ref[...] = jnp.maximum(m, 0.0)


def _head_mean(acc, den0, den1, selfw, bvec):
    blk = 2000
    return pl.pallas_call(
        _mean_body,
        grid=(N // blk,),
        in_specs=[
            pl.BlockSpec((blk, F), lambda i: (i, 0)),
            pl.BlockSpec((blk, 1), lambda i: (i, 0)),
            pl.BlockSpec((blk, 1), lambda i: (i, 0)),
            pl.BlockSpec((blk, 2), lambda i: (i, 0)),
            pl.BlockSpec((1, HID), lambda i: (0, 0)),
        ],
        out_specs=pl.BlockSpec((blk, HID), lambda i: (i, 0)),
        out_shape=jax.ShapeDtypeStruct((N, HID), _f32),
    )(acc, den0.reshape(N, 1), den1.reshape(N, 1), selfw, bvec)


def _final_body(h_ref, batch_ref, t_ref, wf1_ref, bf1_ref, wf2_ref, bf2_ref,
                out_ref):
    h = h_ref[...]
    z1 = jnp.dot(h, wf1_ref[...][:HID, :], preferred_element_type=_f32)
    bb = batch_ref[...]
    gids = lax.broadcasted_iota(_i32, (bb.shape[0], G), 1)
    oh = (bb == gids).astype(_f32)
    z2 = jnp.dot(oh, t_ref[...], preferred_element_type=_f32)
    z = jnp.maximum(z1 + z2 + bf1_ref[...], 0.0)
    out_ref[...] = jnp.dot(z, wf2_ref[...], preferred_element_type=_f32) \
        + bf2_ref[...]


def _final(hmean, batch2d, t, wf1, bf1v, wf2, bf2v):
    blk = 2000
    return pl.pallas_call(
        _final_body,
        grid=(N // blk,),
        in_specs=[
            pl.BlockSpec((blk, HID), lambda i: (i, 0)),
            pl.BlockSpec((blk, 1), lambda i: (i, 0)),
            pl.BlockSpec((G, HID), lambda i: (0, 0)),
            pl.BlockSpec((F, HID), lambda i: (0, 0)),
            pl.BlockSpec((1, HID), lambda i: (0, 0)),
            pl.BlockSpec((HID, 4), lambda i: (0, 0)),
            pl.BlockSpec((1, 4), lambda i: (0, 0)),
        ],
        out_specs=pl.BlockSpec((blk, 4), lambda i: (i, 0)),
        out_shape=jax.ShapeDtypeStruct((N, 4), _f32),
    )(hmean, batch2d, t, wf1, bf1v, wf2, bf2v)


def _t_body(c_ref, wc_ref, bc_ref, wf1_ref, t_ref):
    c = jnp.maximum(
        jnp.dot(c_ref[...], wc_ref[...], preferred_element_type=_f32)
        + bc_ref[...], 0.0)
    t_ref[...] = jnp.dot(c, wf1_ref[...][HID:, :], preferred_element_type=_f32)


def _climber_t(climber, wc, bcv, wf1):
    return pl.pallas_call(
        _t_body,
        out_shape=jax.ShapeDtypeStruct((G, HID), _f32),
    )(climber, wc, bcv, wf1)


def _bucketize_body(src_hbm, dst_hbm, bsrc_hbm, bdst_hbm, cnt_hbm,
                    svm, dvm, cvm, *bufs):
    sbufs = bufs[:B]
    dbufs = bufs[B:]
    wid = lax.axis_index("s") * NC + lax.axis_index("c")
    lane = lax.iota(_i32, 16)

    for b in range(B):
        psrc0 = (wid * 16 + lane) * 97 + b * 131
        pdst = jnp.broadcast_to(b * RPB + RPB + (wid % 16), (16,))

        def _fill(j, _, b=b, psrc0=psrc0, pdst=pdst):
            sbufs[b][pl.ds(j * 16, 16)] = psrc0 + j * 7
            dbufs[b][pl.ds(j * 16, 16)] = pdst
            return 0
        lax.fori_loop(0, CAPA // 16, _fill, 0)

    def step_body(step, pos):
        base = wid * EPW + step * CH
        pltpu.sync_copy(src_hbm.at[pl.ds(base, CH)], svm)
        pltpu.sync_copy(dst_hbm.at[pl.ds(base, CH)], dvm)

        def it_body(i, pos):
            sv = svm[pl.ds(i * 16, 16)]
            dv = dvm[pl.ds(i * 16, 16)]
            newpos = []
            for b in range(B):
                m = (dv >= b * RPB) & (dv < (b + 1) * RPB)
                cs = jnp.cumsum(m.astype(_i32))
                idx = pos[b] + cs - 1
                plsc.store_scatter(sbufs[b], [idx], sv, mask=m)
                plsc.store_scatter(dbufs[b], [idx], dv, mask=m)
                newpos.append(pos[b] + plsc.all_reduce_population_count(m))
            return tuple(newpos)
        return lax.fori_loop(0, CH // 16, it_body, pos)

    pos = lax.fori_loop(0, NSTEP, step_body,
                        tuple(jnp.zeros((16,), _i32) for _ in range(B)))

    cvecA = jnp.zeros((16,), _i32)
    cvecB = jnp.zeros((16,), _i32)
    for b in range(B):
        cpad = (pos[b] + (2 * K - 1)) & ~(2 * K - 1)
        if b < 16:
            cvecA = jnp.where(lane == b, cpad, cvecA)
        else:
            cvecB = jnp.where(lane == b - 16, cpad, cvecB)
    cvm[pl.ds(0, 16)] = cvecA
    cvm[pl.ds(16, 16)] = cvecB
    pltpu.sync_copy(cvm, cnt_hbm.at[wid])
    for b in range(B):
        cell = (b * NW + wid) * CAPW
        pltpu.sync_copy(sbufs[b].at[pl.ds(0, CAPW)],
                        bsrc_hbm.at[pl.ds(cell, CAPW)])
        pltpu.sync_copy(dbufs[b].at[pl.ds(0, CAPW)],
                        bdst_hbm.at[pl.ds(cell, CAPW)])


def _bucketize(src, dst):
    mesh = plsc.VectorSubcoreMesh(core_axis_name="c", subcore_axis_name="s")
    scratch = [
        pltpu.VMEM((CH,), _i32),
        pltpu.VMEM((CH,), _i32),
        pltpu.VMEM((32,), _i32),
    ] + [pltpu.VMEM((CAPA,), _i32) for _ in range(2 * B)]
    return pl.kernel(
        _bucketize_body,
        out_type=[
            jax.ShapeDtypeStruct((B * NW * CAPW,), _i32),
            jax.ShapeDtypeStruct((B * NW * CAPW,), _i32),
            jax.ShapeDtypeStruct((NW, 32), _i32),
        ],
        mesh=mesh,
        scratch_types=scratch,
        compiler_params=pltpu.CompilerParams(needs_layout_passes=False),
    )(src, dst)


RPT = 312
TAIL = RPB - NS * RPT


def _agg_body(feat_hbm, as0_hbm, as1_hbm, ad0_hbm, ad1_hbm, self_hbm,
              bsrc_hbm, bdst_hbm, cnt_hbm,
              acc_hbm, den0_hbm, den1_hbm, dscr_hbm,
              acc_sh, srcv, dstv, dlv, fbuf, mbuf, s0b, s1b,
              srcv2, dstv2, dlv2, fbuf2, s0b2, s1b2,
              ad0v, ad1v, den0v, den1v, dtmp, dacc, dtmp2, dacc2, cvm,
              sem0, sem1, sem2, sem3, sem4, sem5):
    c = lax.axis_index("c")
    s = lax.axis_index("s")
    lane = lax.iota(_i32, 16)
    z16 = jnp.zeros((16,), _f32)

    pltpu.sync_copy(cnt_hbm, cvm)

    def bucket_body(k, _):
        b = NC * k + c
        lo = b * RPB
        pltpu.sync_copy(ad0_hbm.at[pl.ds(lo, SROWS)],
                        ad0v.at[pl.ds(0, SROWS)])
        pltpu.sync_copy(ad1_hbm.at[pl.ds(lo, SROWS)],
                        ad1v.at[pl.ds(0, SROWS)])

        def _zero(i, _):
            den0v[pl.ds(i * 16, 16)] = z16
            den1v[pl.ds(i * 16, 16)] = z16
            return 0
        lax.fori_loop(0, 5024 // 16, _zero, 0)

        pltpu.sync_copy(
            self_hbm.at[pl.ds(lo + s * RPT, RPT)],
            acc_sh.at[pl.ds(s * RPT, RPT)])

        @pl.when(s == NS - 1)
        def _():
            pltpu.sync_copy(
                self_hbm.at[pl.ds(lo + NS * RPT, TAIL)],
                acc_sh.at[pl.ds(NS * RPT, TAIL)])
        plsc.subcore_barrier()

        srcs = (srcv, srcv2)
        dsts = (dstv, dstv2)
        dls = (dlv, dlv2)
        fbufs = (fbuf, fbuf2)
        mbufs = (mbuf, mbuf)
        s0s = (s0b, s0b2)
        s1s = (s1b, s1b2)
        semf = (sem0, sem3)
        sems0 = (sem1, sem4)
        sems1 = (sem2, sem5)

        for j in range(2):
            w = 2 * s + j
            crowA = cvm[w, pl.ds(0, 16)]
            crowB = cvm[w, pl.ds(16, 16)]
            cnt = jnp.sum(jnp.where(lane == b, crowA, 0)
                          + jnp.where(lane == b - 16, crowB, 0))
            cell = (b * NW + w) * CAPW
            npair = cnt // (2 * K)

            def _start(p, ci, cell=cell):
                off = cell + ci * K
                pltpu.sync_copy(bsrc_hbm.at[pl.ds(off, K)], srcs[p])
                pltpu.sync_copy(bdst_hbm.at[pl.ds(off, K)], dsts[p])
                pltpu.async_copy(feat_hbm.at[srcs[p]], fbufs[p], semf[p])
                pltpu.async_copy(as0_hbm.at[srcs[p]], s0s[p], sems0[p])
                pltpu.async_copy(as1_hbm.at[srcs[p]], s1s[p], sems1[p])

            def _wait(p):
                pltpu.make_async_copy(feat_hbm.at[srcs[p]], fbufs[p],
                                      semf[p]).wait()
                pltpu.make_async_copy(as0_hbm.at[srcs[p]], s0s[p],
                                      sems0[p]).wait()
                pltpu.make_async_copy(as1_hbm.at[srcs[p]], s1s[p],
                                      sems1[p]).wait()

            def _compute(p, lo=lo):
                dlp = dls[p]
                fbp = fbufs[p]
                mbp = mbufs[p]
                for q in range(K // 16):
                    dlp[pl.ds(q * 16, 16)] = \
                        dsts[p][pl.ds(q * 16, 16)] - lo

                def grp_body(g, _):
                    dlg = dlp[pl.ds(g * 16, 16)]
                    a0 = plsc.load_gather(ad0v, [dlg])
                    a1 = plsc.load_gather(ad1v, [dlg])
                    w0v = jnp.exp(_lrelu(s0s[p][pl.ds(g * 16, 16)] + a0))
                    w1v = jnp.exp(_lrelu(s1s[p][pl.ds(g * 16, 16)] + a1))
                    plsc.addupdate_scatter(den0v, [dlg], w0v)
                    plsc.addupdate_scatter(den1v, [dlg], w1v)
                    for j2 in range(16):
                        w0 = w0v[j2]
                        w1 = w1v[j2]
                        i = g * 16 + j2
                        for q in range(4):
                            mbp[i, pl.ds(q * 16, 16)] = \
                                w0 * fbp[i, pl.ds(q * 16, 16)]
                        for q in range(4, 8):
                            mbp[i, pl.ds(q * 16, 16)] = \
                                w1 * fbp[i, pl.ds(q * 16, 16)]
                    return 0
                lax.fori_loop(0, K // 16, grp_body, 0)
                pltpu.sync_copy(mbp, acc_sh.at[dlp], add=True)

            _start(0, 0)

            def pair_body(pi, _, npair_=None):
                ci = pi * 2
                _wait(0)
                _start(1, ci + 1)
                _compute(0)
                _wait(1)

                @pl.when(pi + 1 < npair)
                def _():
                    _start(0, ci + 2)
                _compute(1)
                return 0
            lax.fori_loop(0, npair, pair_body, 0)

            @pl.when(npair == 0)
            def _():
                _wait(0)

        dbase = c * 2 * NS * SROWS
        for h in range(2):
            dv = den0v if h == 0 else den1v
            dhbm = den0_hbm if h == 0 else den1_hbm
            hbase = dbase + h * NS * SROWS
            pltpu.sync_copy(dv.at[pl.ds(0, SROWS)],
                            dscr_hbm.at[pl.ds(hbase + s * SROWS, SROWS)])
        plsc.subcore_barrier()
        for h in range(2):
            dhbm = den0_hbm if h == 0 else den1_hbm
            hbase = dbase + h * NS * SROWS

            def _z2(i, _):
                dacc[pl.ds(i * 16, 16)] = z16
                return 0
            lax.fori_loop(0, 320 // 16, _z2, 0)
            for t in range(NS):
                pltpu.sync_copy(
                    dscr_hbm.at[pl.ds(hbase + t * SROWS + s * RPT, RPT)],
                    dtmp.at[pl.ds(0, RPT)])

                def _acc(i, _):
                    dacc[pl.ds(i * 16, 16)] = (dacc[pl.ds(i * 16, 16)]
                                               + dtmp[pl.ds(i * 16, 16)])
                    return 0
                lax.fori_loop(0, 320 // 16, _acc, 0)
            pltpu.sync_copy(dacc.at[pl.ds(0, RPT)],
                            dhbm.at[pl.ds(lo + s * RPT, RPT)])

            @pl.when(s == NS - 1)
            def _():
                dacc2[...] = z16
                for t in range(NS):
                    pltpu.sync_copy(
                        dscr_hbm.at[pl.ds(hbase + t * SROWS + NS * RPT,
                                          TAIL)],
                        dtmp2.at[pl.ds(0, TAIL)])
                    dacc2[...] = dacc2[...] + dtmp2[...]
                pltpu.sync_copy(dacc2.at[pl.ds(0, TAIL)],
                                dhbm.at[pl.ds(lo + NS * RPT, TAIL)])

        pltpu.sync_copy(
            acc_sh.at[pl.ds(s * RPT, RPT)],
            acc_hbm.at[pl.ds(lo + s * RPT, RPT)])

        @pl.when(s == NS - 1)
        def _():
            pltpu.sync_copy(
                acc_sh.at[pl.ds(NS * RPT, TAIL)],
                acc_hbm.at[pl.ds(lo + NS * RPT, TAIL)])
        plsc.subcore_barrier()
        return 0

    lax.fori_loop(0, B // NC, bucket_body, 0)


def _aggregate(feat, as0, as1, ad0, ad1, selfinit, bsrc, bdst, cnt):
    mesh = plsc.VectorSubcoreMesh(core_axis_name="c", subcore_axis_name="s")
    scratch = [
        pltpu.VMEM_SHARED((SROWS, F), _f32),
        pltpu.VMEM((K,), _i32),
        pltpu.VMEM((K,), _i32),
        pltpu.VMEM((K,), _i32),
        pltpu.VMEM((K, F), _f32),
        pltpu.VMEM((K, F), _f32),
        pltpu.VMEM((K,), _f32),
        pltpu.VMEM((K,), _f32),
        pltpu.VMEM((K,), _i32),
        pltpu.VMEM((K,), _i32),
        pltpu.VMEM((K,), _i32),
        pltpu.VMEM((K, F), _f32),
        pltpu.VMEM((K,), _f32),
        pltpu.VMEM((K,), _f32),
        pltpu.VMEM((5024,), _f32),
        pltpu.VMEM((5024,), _f32),
        pltpu.VMEM((5024,), _f32),
        pltpu.VMEM((5024,), _f32),
        pltpu.VMEM((320,), _f32),
        pltpu.VMEM((320,), _f32),
        pltpu.VMEM((16,), _f32),
        pltpu.VMEM((16,), _f32),
        pltpu.VMEM((NW, 32), _i32),
        pltpu.SemaphoreType.DMA,
        pltpu.SemaphoreType.DMA,
        pltpu.SemaphoreType.DMA,
        pltpu.SemaphoreType.DMA,
        pltpu.SemaphoreType.DMA,
        pltpu.SemaphoreType.DMA,
    ]
    return pl.kernel(
        _agg_body,
        out_type=[
            jax.ShapeDtypeStruct((N, F), _f32),
            jax.ShapeDtypeStruct((N,), _f32),
            jax.ShapeDtypeStruct((N,), _f32),
            jax.ShapeDtypeStruct((NC * 2 * NS * SROWS,), _f32),
        ],
        mesh=mesh,
        scratch_types=scratch,
        compiler_params=pltpu.CompilerParams(needs_layout_passes=False),
    )(feat, as0, as1, ad0, ad1, selfinit, bsrc, bdst, cnt)


def kernel(x, edge_index, batch, climber, W1, a_src1, a_dst1, b1,
           W2, a_src2, a_dst2, b2, Wc, bc, Wf1, bf1, Wf2, bf2):
    src = edge_index[0]
    dst = edge_index[1]
    bsrc, bdst, cnt = _bucketize(src, dst)

    def _layer(hin, w, a_s, a_d, bvec):
        feat, as0, as1, ad0, ad1, selfw, selfi = _prep(
            hin, w, a_s.reshape(2, HID), a_d.reshape(2, HID))
        as0 = as0.reshape(N)
        as1 = as1.reshape(N)
        ad0 = jnp.pad(ad0.reshape(N), (0, TRASH))
        ad1 = jnp.pad(ad1.reshape(N), (0, TRASH))
        acc, den0, den1, _ = _aggregate(feat, as0, as1, ad0, ad1, selfi,
                                        bsrc, bdst, cnt)
        return _head_mean(acc, den0, den1, selfw, bvec.reshape(1, HID))

    h1 = _layer(x, W1, a_src1, a_dst1, b1)
    h2 = _layer(h1, W2, a_src2, a_dst2, b2)

    t = _climber_t(climber, Wc, bc.reshape(1, HID), Wf1)
    return _final(h2, batch.reshape(N, 1), t, Wf1, bf1.reshape(1, HID),
                  Wf2, bf2.reshape(1, 4))


# speedup vs baseline: 120.9578x; 1.1456x over previous
import jax
import jax.numpy as jnp
from jax import lax
from jax.experimental import pallas as pl
from jax.experimental.pallas import tpu as pltpu
from jax.experimental.pallas import tpu_sc as plsc

N = 100000
E = 1600000
G = 1024
HID = 64
F = 2 * HID
B = 20
RPB = N // B
TRASH = 16
SROWS = RPB + TRASH
NC, NS = 2, 16
NW = NC * NS
EPW = E // NW
CH = 1000
NSTEP = EPW // CH
CAPW = 3072
CAPA = CAPW + 16
K = 128

_f32 = jnp.float32
_i32 = jnp.int32


def _lrelu(v):
    return jnp.where(v >= 0, v, 0.2 * v)


def _prep_body(hin_ref, w_ref, as_ref, ad_ref, feat_ref, as0_ref, as1_ref,
               ad0_ref, ad1_ref, selfw_ref, self_ref):
    h = jnp.dot(hin_ref[...], w_ref[...], preferred_element_type=_f32)
    a_s = as_ref[...]
    a_d = ad_ref[...]
    as0 = jnp.sum(h[:, :HID] * a_s[0:1, :], axis=1, keepdims=True)
    as1 = jnp.sum(h[:, HID:] * a_s[1:2, :], axis=1, keepdims=True)
    ad0 = jnp.sum(h[:, :HID] * a_d[0:1, :], axis=1, keepdims=True)
    ad1 = jnp.sum(h[:, HID:] * a_d[1:2, :], axis=1, keepdims=True)
    feat_ref[...] = h
    as0_ref[...] = as0
    as1_ref[...] = as1
    ad0_ref[...] = ad0
    ad1_ref[...] = ad1
    w0 = jnp.exp(_lrelu(as0 + ad0))
    w1 = jnp.exp(_lrelu(as1 + ad1))
    selfw_ref[...] = jnp.concatenate([w0, w1], axis=1)
    self_ref[...] = jnp.concatenate(
        [h[:, :HID] * w0, h[:, HID:] * w1], axis=1)


def _prep(hin, w, a_s, a_d):
    blk = 2000
    grid = N // blk
    kin = hin.shape[1]
    return pl.pallas_call(
        _prep_body,
        grid=(grid,),
        in_specs=[
            pl.BlockSpec((blk, kin), lambda i: (i, 0)),
            pl.BlockSpec((kin, F), lambda i: (0, 0)),
            pl.BlockSpec((2, HID), lambda i: (0, 0)),
            pl.BlockSpec((2, HID), lambda i: (0, 0)),
        ],
        out_specs=[
            pl.BlockSpec((blk, F), lambda i: (i, 0)),
            pl.BlockSpec((blk, 1), lambda i: (i, 0)),
            pl.BlockSpec((blk, 1), lambda i: (i, 0)),
            pl.BlockSpec((blk, 1), lambda i: (i, 0)),
            pl.BlockSpec((blk, 1), lambda i: (i, 0)),
            pl.BlockSpec((blk, 2), lambda i: (i, 0)),
            pl.BlockSpec((blk, F), lambda i: (i, 0)),
        ],
        out_shape=[
            jax.ShapeDtypeStruct((N, F), _f32),
            jax.ShapeDtypeStruct((N, 1), _f32),
            jax.ShapeDtypeStruct((N, 1), _f32),
            jax.ShapeDtypeStruct((N, 1), _f32),
            jax.ShapeDtypeStruct((N, 1), _f32),
            jax.ShapeDtypeStruct((N, 2), _f32),
            jax.ShapeDtypeStruct((N, F), _f32),
        ],
    )(hin, w, a_s, a_d)


def _mean_body(acc_ref, den0_ref, den1_ref, selfw_ref, b_ref, out_ref):
    acc = acc_ref[...]
    selfw = selfw_ref[...]
    den0 = den0_ref[...] + selfw[:, 0:1] + 1e-16
    den1 = den1_ref[...] + selfw[:, 1:2] + 1e-16
    m = 0.5 * (acc[:, :HID] / den0 + acc[:, HID:F] / den1) + b_ref[...]
    out_ref[...] = jnp.maximum(m, 0.0)


def _head_mean(acc, den0, den1, selfw, bvec):
    blk = 2000
    return pl.pallas_call(
        _mean_body,
        grid=(N // blk,),
        in_specs=[
            pl.BlockSpec((blk, F), lambda i: (i, 0)),
            pl.BlockSpec((blk, 1), lambda i: (i, 0)),
            pl.BlockSpec((blk, 1), lambda i: (i, 0)),
            pl.BlockSpec((blk, 2), lambda i: (i, 0)),
            pl.BlockSpec((1, HID), lambda i: (0, 0)),
        ],
        out_specs=pl.BlockSpec((blk, HID), lambda i: (i, 0)),
        out_shape=jax.ShapeDtypeStruct((N, HID), _f32),
    )(acc, den0.reshape(N, 1), den1.reshape(N, 1), selfw, bvec)


def _final_body(h_ref, batch_ref, t_ref, wf1_ref, bf1_ref, wf2_ref, bf2_ref,
                out_ref):
    h = h_ref[...]
    z1 = jnp.dot(h, wf1_ref[...][:HID, :], preferred_element_type=_f32)
    bb = batch_ref[...]
    gids = lax.broadcasted_iota(_i32, (bb.shape[0], G), 1)
    oh = (bb == gids).astype(_f32)
    z2 = jnp.dot(oh, t_ref[...], preferred_element_type=_f32)
    z = jnp.maximum(z1 + z2 + bf1_ref[...], 0.0)
    out_ref[...] = jnp.dot(z, wf2_ref[...], preferred_element_type=_f32) \
        + bf2_ref[...]


def _final(hmean, batch2d, t, wf1, bf1v, wf2, bf2v):
    blk = 2000
    return pl.pallas_call(
        _final_body,
        grid=(N // blk,),
        in_specs=[
            pl.BlockSpec((blk, HID), lambda i: (i, 0)),
            pl.BlockSpec((blk, 1), lambda i: (i, 0)),
            pl.BlockSpec((G, HID), lambda i: (0, 0)),
            pl.BlockSpec((F, HID), lambda i: (0, 0)),
            pl.BlockSpec((1, HID), lambda i: (0, 0)),
            pl.BlockSpec((HID, 4), lambda i: (0, 0)),
            pl.BlockSpec((1, 4), lambda i: (0, 0)),
        ],
        out_specs=pl.BlockSpec((blk, 4), lambda i: (i, 0)),
        out_shape=jax.ShapeDtypeStruct((N, 4), _f32),
    )(hmean, batch2d, t, wf1, bf1v, wf2, bf2v)


def _t_body(c_ref, wc_ref, bc_ref, wf1_ref, t_ref):
    c = jnp.maximum(
        jnp.dot(c_ref[...], wc_ref[...], preferred_element_type=_f32)
        + bc_ref[...], 0.0)
    t_ref[...] = jnp.dot(c, wf1_ref[...][HID:, :], preferred_element_type=_f32)


def _climber_t(climber, wc, bcv, wf1):
    return pl.pallas_call(
        _t_body,
        out_shape=jax.ShapeDtypeStruct((G, HID), _f32),
    )(climber, wc, bcv, wf1)


def _bucketize_body(src_hbm, dst_hbm, bsrc_hbm, bdst_hbm, cnt_hbm,
                    svm, dvm, cvm, *bufs):
    sbufs = bufs[:B]
    dbufs = bufs[B:]
    wid = lax.axis_index("s") * NC + lax.axis_index("c")
    lane = lax.iota(_i32, 16)

    for b in range(B):
        psrc0 = (wid * 16 + lane) * 97 + b * 131
        pdst = jnp.broadcast_to(b * RPB + RPB + (wid % 16), (16,))

        def _fill(j, _, b=b, psrc0=psrc0, pdst=pdst):
            sbufs[b][pl.ds(j * 16, 16)] = psrc0 + j * 7
            dbufs[b][pl.ds(j * 16, 16)] = pdst
            return 0
        lax.fori_loop(0, CAPA // 16, _fill, 0)

    def step_body(step, pos):
        base = wid * EPW + step * CH
        pltpu.sync_copy(src_hbm.at[pl.ds(base, CH)], svm)
        pltpu.sync_copy(dst_hbm.at[pl.ds(base, CH)], dvm)

        def it_body(i, pos):
            sv = svm[pl.ds(i * 16, 16)]
            dv = dvm[pl.ds(i * 16, 16)]
            newpos = []
            for b in range(B):
                m = (dv >= b * RPB) & (dv < (b + 1) * RPB)
                cs = jnp.cumsum(m.astype(_i32))
                idx = pos[b] + cs - 1
                plsc.store_scatter(sbufs[b], [idx], sv, mask=m)
                plsc.store_scatter(dbufs[b], [idx], dv, mask=m)
                newpos.append(pos[b] + plsc.all_reduce_population_count(m))
            return tuple(newpos)
        return lax.fori_loop(0, CH // 16, it_body, pos)

    pos = lax.fori_loop(0, NSTEP, step_body,
                        tuple(jnp.zeros((16,), _i32) for _ in range(B)))

    cvecA = jnp.zeros((16,), _i32)
    cvecB = jnp.zeros((16,), _i32)
    for b in range(B):
        cpad = (pos[b] + (2 * K - 1)) & ~(2 * K - 1)
        if b < 16:
            cvecA = jnp.where(lane == b, cpad, cvecA)
        else:
            cvecB = jnp.where(lane == b - 16, cpad, cvecB)
    cvm[pl.ds(0, 16)] = cvecA
    cvm[pl.ds(16, 16)] = cvecB
    pltpu.sync_copy(cvm, cnt_hbm.at[wid])
    for b in range(B):
        cell = (b * NW + wid) * CAPW
        pltpu.sync_copy(sbufs[b].at[pl.ds(0, CAPW)],
                        bsrc_hbm.at[pl.ds(cell, CAPW)])
        pltpu.sync_copy(dbufs[b].at[pl.ds(0, CAPW)],
                        bdst_hbm.at[pl.ds(cell, CAPW)])


def _bucketize(src, dst):
    mesh = plsc.VectorSubcoreMesh(core_axis_name="c", subcore_axis_name="s")
    scratch = [
        pltpu.VMEM((CH,), _i32),
        pltpu.VMEM((CH,), _i32),
        pltpu.VMEM((32,), _i32),
    ] + [pltpu.VMEM((CAPA,), _i32) for _ in range(2 * B)]
    return pl.kernel(
        _bucketize_body,
        out_type=[
            jax.ShapeDtypeStruct((B * NW * CAPW,), _i32),
            jax.ShapeDtypeStruct((B * NW * CAPW,), _i32),
            jax.ShapeDtypeStruct((NW, 32), _i32),
        ],
        mesh=mesh,
        scratch_types=scratch,
        compiler_params=pltpu.CompilerParams(needs_layout_passes=False),
    )(src, dst)


RPT = 312
TAIL = RPB - NS * RPT


def _agg_body(feat_hbm, as0_hbm, as1_hbm, ad0_hbm, ad1_hbm, self_hbm,
              bsrc_hbm, bdst_hbm, cnt_hbm,
              acc_hbm, den0_hbm, den1_hbm, dscr_hbm,
              acc_sh, srcv, dstv, dlv, fbuf, mbuf, s0b, s1b,
              srcv2, dstv2, dlv2, fbuf2, s0b2, s1b2,
              ad0v, ad1v, den0v, den1v, dred, dacc, dacc2, cvm,
              sem0, sem1, sem2, sem3, sem4, sem5, semd):
    c = lax.axis_index("c")
    s = lax.axis_index("s")
    lane = lax.iota(_i32, 16)
    z16 = jnp.zeros((16,), _f32)

    pltpu.sync_copy(cnt_hbm, cvm)

    def bucket_body(k, _):
        b = NC * k + c
        lo = b * RPB
        pltpu.sync_copy(ad0_hbm.at[pl.ds(lo, SROWS)],
                        ad0v.at[pl.ds(0, SROWS)])
        pltpu.sync_copy(ad1_hbm.at[pl.ds(lo, SROWS)],
                        ad1v.at[pl.ds(0, SROWS)])

        def _zero(i, _):
            den0v[pl.ds(i * 16, 16)] = z16
            den1v[pl.ds(i * 16, 16)] = z16
            return 0
        lax.fori_loop(0, 5024 // 16, _zero, 0)

        pltpu.sync_copy(
            self_hbm.at[pl.ds(lo + s * RPT, RPT)],
            acc_sh.at[pl.ds(s * RPT, RPT)])

        @pl.when(s == NS - 1)
        def _():
            pltpu.sync_copy(
                self_hbm.at[pl.ds(lo + NS * RPT, TAIL)],
                acc_sh.at[pl.ds(NS * RPT, TAIL)])
        plsc.subcore_barrier()

        srcs = (srcv, srcv2)
        dsts = (dstv, dstv2)
        dls = (dlv, dlv2)
        fbufs = (fbuf, fbuf2)
        mbufs = (mbuf, mbuf)
        s0s = (s0b, s0b2)
        s1s = (s1b, s1b2)
        semf = (sem0, sem3)
        sems0 = (sem1, sem4)
        sems1 = (sem2, sem5)

        for j in range(2):
            w = 2 * s + j
            crowA = cvm[w, pl.ds(0, 16)]
            crowB = cvm[w, pl.ds(16, 16)]
            cnt = jnp.sum(jnp.where(lane == b, crowA, 0)
                          + jnp.where(lane == b - 16, crowB, 0))
            cell = (b * NW + w) * CAPW
            npair = cnt // (2 * K)

            def _start(p, ci, cell=cell):
                off = cell + ci * K
                pltpu.sync_copy(bsrc_hbm.at[pl.ds(off, K)], srcs[p])
                pltpu.sync_copy(bdst_hbm.at[pl.ds(off, K)], dsts[p])
                pltpu.async_copy(feat_hbm.at[srcs[p]], fbufs[p], semf[p])
                pltpu.async_copy(as0_hbm.at[srcs[p]], s0s[p], sems0[p])
                pltpu.async_copy(as1_hbm.at[srcs[p]], s1s[p], sems1[p])

            def _wait(p):
                pltpu.make_async_copy(feat_hbm.at[srcs[p]], fbufs[p],
                                      semf[p]).wait()
                pltpu.make_async_copy(as0_hbm.at[srcs[p]], s0s[p],
                                      sems0[p]).wait()
                pltpu.make_async_copy(as1_hbm.at[srcs[p]], s1s[p],
                                      sems1[p]).wait()

            def _compute(p, lo=lo):
                dlp = dls[p]
                fbp = fbufs[p]
                mbp = mbufs[p]
                for q in range(K // 16):
                    dlp[pl.ds(q * 16, 16)] = \
                        dsts[p][pl.ds(q * 16, 16)] - lo

                def grp_body(g, _):
                    dlg = dlp[pl.ds(g * 16, 16)]
                    a0 = plsc.load_gather(ad0v, [dlg])
                    a1 = plsc.load_gather(ad1v, [dlg])
                    w0v = jnp.exp(_lrelu(s0s[p][pl.ds(g * 16, 16)] + a0))
                    w1v = jnp.exp(_lrelu(s1s[p][pl.ds(g * 16, 16)] + a1))
                    plsc.addupdate_scatter(den0v, [dlg], w0v)
                    plsc.addupdate_scatter(den1v, [dlg], w1v)
                    for j2 in range(16):
                        w0 = w0v[j2]
                        w1 = w1v[j2]
                        i = g * 16 + j2
                        for q in range(4):
                            mbp[i, pl.ds(q * 16, 16)] = \
                                w0 * fbp[i, pl.ds(q * 16, 16)]
                        for q in range(4, 8):
                            mbp[i, pl.ds(q * 16, 16)] = \
                                w1 * fbp[i, pl.ds(q * 16, 16)]
                    return 0
                lax.fori_loop(0, K // 16, grp_body, 0)
                pltpu.sync_copy(mbp, acc_sh.at[dlp], add=True)

            _start(0, 0)

            def pair_body(pi, _, npair_=None):
                ci = pi * 2
                _wait(0)
                _start(1, ci + 1)
                _compute(0)
                _wait(1)

                @pl.when(pi + 1 < npair)
                def _():
                    _start(0, ci + 2)
                _compute(1)
                return 0
            lax.fori_loop(0, npair, pair_body, 0)

            @pl.when(npair == 0)
            def _():
                _wait(0)

        dbase = c * 2 * NS * SROWS
        for h in range(2):
            dv = den0v if h == 0 else den1v
            dhbm = den0_hbm if h == 0 else den1_hbm
            hbase = dbase + h * NS * SROWS
            pltpu.sync_copy(dv.at[pl.ds(0, SROWS)],
                            dscr_hbm.at[pl.ds(hbase + s * SROWS, SROWS)])
        plsc.subcore_barrier()
        for h in range(2):
            dhbm = den0_hbm if h == 0 else den1_hbm
            hbase = dbase + h * NS * SROWS

            for t in range(NS):
                pltpu.async_copy(
                    dscr_hbm.at[pl.ds(hbase + t * SROWS + s * RPT, RPT)],
                    dred.at[pl.ds(t * 320, RPT)], semd)
            for t in range(NS):
                pltpu.make_async_copy(
                    dscr_hbm.at[pl.ds(hbase + t * SROWS + s * RPT, RPT)],
                    dred.at[pl.ds(t * 320, RPT)], semd).wait()

            def _acc(g, _):
                v = dred[pl.ds(g * 16, 16)]
                for t in range(1, NS):
                    v = v + dred[pl.ds(t * 320 + g * 16, 16)]
                dacc[pl.ds(g * 16, 16)] = v
                return 0
            lax.fori_loop(0, 320 // 16, _acc, 0)
            pltpu.sync_copy(dacc.at[pl.ds(0, RPT)],
                            dhbm.at[pl.ds(lo + s * RPT, RPT)])

            @pl.when(s == NS - 1)
            def _():
                for t in range(NS):
                    pltpu.async_copy(
                        dscr_hbm.at[pl.ds(hbase + t * SROWS + NS * RPT,
                                          TAIL)],
                        dred.at[pl.ds(t * 320, TAIL)], semd)
                for t in range(NS):
                    pltpu.make_async_copy(
                        dscr_hbm.at[pl.ds(hbase + t * SROWS + NS * RPT,
                                          TAIL)],
                        dred.at[pl.ds(t * 320, TAIL)], semd).wait()
                v2 = dred[pl.ds(0, 16)]
                for t in range(1, NS):
                    v2 = v2 + dred[pl.ds(t * 320, 16)]
                dacc2[...] = v2
                pltpu.sync_copy(dacc2.at[pl.ds(0, TAIL)],
                                dhbm.at[pl.ds(lo + NS * RPT, TAIL)])

        pltpu.sync_copy(
            acc_sh.at[pl.ds(s * RPT, RPT)],
            acc_hbm.at[pl.ds(lo + s * RPT, RPT)])

        @pl.when(s == NS - 1)
        def _():
            pltpu.sync_copy(
                acc_sh.at[pl.ds(NS * RPT, TAIL)],
                acc_hbm.at[pl.ds(lo + NS * RPT, TAIL)])
        plsc.subcore_barrier()
        return 0

    lax.fori_loop(0, B // NC, bucket_body, 0)


def _aggregate(feat, as0, as1, ad0, ad1, selfinit, bsrc, bdst, cnt):
    mesh = plsc.VectorSubcoreMesh(core_axis_name="c", subcore_axis_name="s")
    scratch = [
        pltpu.VMEM_SHARED((SROWS, F), _f32),
        pltpu.VMEM((K,), _i32),
        pltpu.VMEM((K,), _i32),
        pltpu.VMEM((K,), _i32),
        pltpu.VMEM((K, F), _f32),
        pltpu.VMEM((K, F), _f32),
        pltpu.VMEM((K,), _f32),
        pltpu.VMEM((K,), _f32),
        pltpu.VMEM((K,), _i32),
        pltpu.VMEM((K,), _i32),
        pltpu.VMEM((K,), _i32),
        pltpu.VMEM((K, F), _f32),
        pltpu.VMEM((K,), _f32),
        pltpu.VMEM((K,), _f32),
        pltpu.VMEM((5024,), _f32),
        pltpu.VMEM((5024,), _f32),
        pltpu.VMEM((5024,), _f32),
        pltpu.VMEM((5024,), _f32),
        pltpu.VMEM((NS * 320,), _f32),
        pltpu.VMEM((320,), _f32),
        pltpu.VMEM((16,), _f32),
        pltpu.VMEM((NW, 32), _i32),
        pltpu.SemaphoreType.DMA,
        pltpu.SemaphoreType.DMA,
        pltpu.SemaphoreType.DMA,
        pltpu.SemaphoreType.DMA,
        pltpu.SemaphoreType.DMA,
        pltpu.SemaphoreType.DMA,
        pltpu.SemaphoreType.DMA,
    ]
    return pl.kernel(
        _agg_body,
        out_type=[
            jax.ShapeDtypeStruct((N, F), _f32),
            jax.ShapeDtypeStruct((N,), _f32),
            jax.ShapeDtypeStruct((N,), _f32),
            jax.ShapeDtypeStruct((NC * 2 * NS * SROWS,), _f32),
        ],
        mesh=mesh,
        scratch_types=scratch,
        compiler_params=pltpu.CompilerParams(needs_layout_passes=False),
    )(feat, as0, as1, ad0, ad1, selfinit, bsrc, bdst, cnt)


def kernel(x, edge_index, batch, climber, W1, a_src1, a_dst1, b1,
           W2, a_src2, a_dst2, b2, Wc, bc, Wf1, bf1, Wf2, bf2):
    src = edge_index[0]
    dst = edge_index[1]
    bsrc, bdst, cnt = _bucketize(src, dst)

    def _layer(hin, w, a_s, a_d, bvec):
        feat, as0, as1, ad0, ad1, selfw, selfi = _prep(
            hin, w, a_s.reshape(2, HID), a_d.reshape(2, HID))
        as0 = as0.reshape(N)
        as1 = as1.reshape(N)
        ad0 = jnp.pad(ad0.reshape(N), (0, TRASH))
        ad1 = jnp.pad(ad1.reshape(N), (0, TRASH))
        acc, den0, den1, _ = _aggregate(feat, as0, as1, ad0, ad1, selfi,
                                        bsrc, bdst, cnt)
        return _head_mean(acc, den0, den1, selfw, bvec.reshape(1, HID))

    h1 = _layer(x, W1, a_src1, a_dst1, b1)
    h2 = _layer(h1, W2, a_src2, a_dst2, b2)

    t = _climber_t(climber, Wc, bc.reshape(1, HID), Wf1)
    return _final(h2, batch.reshape(N, 1), t, Wf1, bf1.reshape(1, HID),
                  Wf2, bf2.reshape(1, 4))
